# Initial kernel scaffold; baseline (speedup 1.0000x reference)
import jax
import jax.numpy as jnp
from jax.experimental import pallas as pl


def kernel(x, edge_index, edge_attr, Wn0, We0, Wfn0, bfn0, Wfe0, bfe0, Wn1, We1, Wfn1, bfn1, Wfe1, bfe1):
    raise NotImplementedError("write your pallas kernel here")


# speedup vs baseline: 1.1593x; 1.1593x over previous
import functools
import math

import jax
import jax.numpy as jnp
from jax import lax
from jax.experimental import pallas as pl
from jax.experimental.pallas import tpu as pltpu
from jax.experimental.pallas import tpu_sc as plsc

NC = 2
NS = 16
NW = NC * NS

B = 128
CH = 80
EW = B * CH
EP = NW * EW
NP = 10240

_BN_SCALE = 1.0 / math.sqrt(1.0 + 1e-5)


def _dot(a, b):
    return jnp.dot(a, b, preferred_element_type=jnp.float32)


def _node_pre_body(x_ref, w_ref, h_ref):
    h_ref[...] = jax.nn.relu(_dot(x_ref[...], w_ref[...]))


def _node_pre(xp, Wn0):
    H2 = Wn0.shape[1]
    return pl.pallas_call(
        _node_pre_body,
        out_shape=jax.ShapeDtypeStruct((NP, H2), jnp.float32),
    )(xp, Wn0)


def _make_sc_gather(H2):
    NBUF = 4
    G = CH // NBUF
    ROWS = NP // NS
    mesh = plsc.VectorSubcoreMesh(core_axis_name="c", subcore_axis_name="s")

    @functools.partial(
        pl.kernel,
        mesh=mesh,
        out_type=(
            jax.ShapeDtypeStruct((NW, CH, B, H2), jnp.float32),
            jax.ShapeDtypeStruct((NW, CH, B, H2), jnp.float32),
        ),
        scratch_types=[
            pltpu.MemorySpace.VMEM_SHARED((NP, H2), jnp.float32),
            pltpu.VMEM((CH, B), jnp.int32),
            pltpu.VMEM((NBUF, B, H2), jnp.float32),
            pltpu.SemaphoreType.DMA,
            pltpu.SemaphoreType.DMA,
        ],
    )
    def gather_k(h0_hbm, src_hbm, dst_hbm, mS_hbm, mD_hbm,
                 h0sh, idx_v, bufs, gsem, wsem):
        s = lax.axis_index("s")
        wid = lax.axis_index("c") * NS + s
        rs = pl.ds(s * ROWS, ROWS)
        pltpu.sync_copy(h0_hbm.at[rs], h0sh.at[rs])
        plsc.subcore_barrier()
        for ix_hbm, out_hbm in ((src_hbm, mS_hbm), (dst_hbm, mD_hbm)):
            pltpu.sync_copy(ix_hbm.at[wid], idx_v)

            def group(g, _):
                gds = []
                for b in range(NBUF):
                    j = g * NBUF + b
                    gds.append(pltpu.async_copy(
                        h0sh.at[idx_v.at[j]], bufs.at[b], gsem))
                for d in gds:
                    d.wait()
                pltpu.async_copy(
                    bufs, out_hbm.at[wid, pl.ds(g * NBUF, NBUF)], wsem).wait()
                return 0

            lax.fori_loop(0, G, group, 0)

    return gather_k


def _edge_mid_body(ea_ref, ms_ref, md_ref, we0_ref, wt_ref, wb_ref, bfe_ref,
                   we1_ref, c01_ref):
    bf = jnp.bfloat16
    c0 = jax.nn.relu(_dot(ea_ref[...].astype(bf), we0_ref[...].astype(bf)))
    m = (ms_ref[...] + md_ref[...]).astype(bf)
    t = jax.nn.relu(_dot(m, wt_ref[...].astype(bf))
                    + _dot(c0.astype(bf), wb_ref[...].astype(bf))
                    + bfe_ref[...])
    c01_ref[...] = jnp.concatenate(
        [c0, jax.nn.relu(_dot(t.astype(bf), we1_ref[...].astype(bf)))],
        axis=-1)


def _edge_mid(eap, mS, mD, We0, Wfe0, bfe0, We1):
    DE = eap.shape[1]
    H2 = We0.shape[1]
    H = Wfe0.shape[1]
    BE = 2560
    grid = (EP // BE,)
    wt = Wfe0[:H2]
    wb = Wfe0[H2:]
    bfe = bfe0.reshape(1, H)
    full = lambda r, c: pl.BlockSpec((r, c), lambda i: (0, 0))
    return pl.pallas_call(
        _edge_mid_body,
        grid=grid,
        in_specs=[
            pl.BlockSpec((BE, DE), lambda i: (i, 0)),
            pl.BlockSpec((BE, H2), lambda i: (i, 0)),
            pl.BlockSpec((BE, H2), lambda i: (i, 0)),
            full(DE, H2),
            full(H2, H),
            full(H2, H),
            full(1, H),
            full(H, H2),
        ],
        out_specs=pl.BlockSpec((BE, 2 * H2), lambda i: (i, 0)),
        out_shape=jax.ShapeDtypeStruct((EP, 2 * H2), jnp.float32),
    )(eap, mS, mD, We0, wt, wb, bfe, We1)


def _make_sc_scatter(H):
    ROWS = NP // NS
    mesh = plsc.VectorSubcoreMesh(core_axis_name="c", subcore_axis_name="s")

    @functools.partial(
        pl.kernel,
        mesh=mesh,
        out_type=jax.ShapeDtypeStruct((NC, NP, H), jnp.float32),
        scratch_types=[
            pltpu.VMEM((CH, B), jnp.int32),
            pltpu.VMEM((2, B, H), jnp.float32),
            pltpu.MemorySpace.VMEM_SHARED((NP, H), jnp.float32),
            pltpu.SemaphoreType.DMA,
            pltpu.SemaphoreType.DMA,
        ],
    )
    def scatter_k(c01_hbm, dst_hbm, z_hbm, acc_hbm, idx_v, bufs, acc,
                  lsem, ssem):
        c = lax.axis_index("c")
        s = lax.axis_index("s")
        wid = c * NS + s
        rs = pl.ds(s * ROWS, ROWS)
        pltpu.sync_copy(dst_hbm.at[wid], idx_v)
        pltpu.sync_copy(z_hbm.at[rs], acc.at[rs])
        plsc.subcore_barrier()
        sdesc = [None, None]
        for j in range(CH):
            b = j % 2
            if sdesc[b] is not None:
                sdesc[b].wait()
            pltpu.async_copy(
                c01_hbm.at[wid * CH + j], bufs.at[b], lsem).wait()
            sdesc[b] = pltpu.async_copy(
                bufs.at[b], acc.at[idx_v.at[j]], ssem, add=True)
        for d in sdesc:
            if d is not None:
                d.wait()
        plsc.subcore_barrier()
        pltpu.sync_copy(acc.at[rs], acc_hbm.at[c, rs])

    return scatter_k


def _node_final_body(ap_ref, h0_ref, w0t_ref, w0b_ref, b0_ref,
                     wn1_ref, w1t_ref, w1b_ref, b1_ref, out_ref):
    H2 = h0_ref.shape[1]
    accs = ap_ref[0] + ap_ref[1]
    aggr0 = accs[:, :H2]
    aggr1 = accs[:, H2:]
    x1 = jax.nn.relu(_dot(aggr0, w0t_ref[...]) + _dot(h0_ref[...], w0b_ref[...])
                     + b0_ref[...]) * _BN_SCALE
    h1 = jax.nn.relu(_dot(x1, wn1_ref[...]))
    out_ref[...] = jax.nn.relu(
        _dot(aggr1, w1t_ref[...]) + _dot(h1, w1b_ref[...])
        + b1_ref[...]) * _BN_SCALE


def _node_final(ap, h0p, Wfn0, bfn0, Wn1, Wfn1, bfn1):
    H2 = Wn1.shape[1]
    H = Wfn0.shape[1]
    return pl.pallas_call(
        _node_final_body,
        out_shape=jax.ShapeDtypeStruct((NP, H), jnp.float32),
    )(ap, h0p, Wfn0[:H2], Wfn0[H2:], bfn0.reshape(1, H),
      Wn1, Wfn1[:H2], Wfn1[H2:], bfn1.reshape(1, H))


def kernel(x, edge_index, edge_attr, Wn0, We0, Wfn0, bfn0, Wfe0, bfe0,
           Wn1, We1, Wfn1, bfn1, Wfe1, bfe1):
    N = x.shape[0]
    E = edge_index.shape[1]
    H2 = Wn0.shape[1]
    H = Wfn0.shape[1]
    DE = edge_attr.shape[1]

    xp = jnp.pad(x, ((0, NP - N), (0, 0)))
    pad_idx = jnp.full((2, EP - E), N, dtype=edge_index.dtype)
    ei = jnp.concatenate([edge_index, pad_idx], axis=1)
    src3 = ei[0].reshape(NW, CH, B)
    dst3 = ei[1].reshape(NW, CH, B)
    eap = jnp.pad(edge_attr, ((0, EP - E), (0, 0)))

    h0p = _node_pre(xp, Wn0)

    mS = h0p[ei[0]]
    mD = h0p[ei[1]]

    c01 = _edge_mid(eap, mS, mD, We0, Wfe0, bfe0, We1)

    zeros = jnp.zeros((NP, H), jnp.float32)
    ap = _make_sc_scatter(H)(c01.reshape(NW * CH, B, H), dst3, zeros)

    out = _node_final(ap, h0p, Wfn0, bfn0, Wn1, Wfn1, bfn1)
    return out[:N]


# speedup vs baseline: 4.7097x; 4.0625x over previous
import functools
import math

import jax
import jax.numpy as jnp
from jax import lax
from jax.experimental import pallas as pl
from jax.experimental.pallas import tpu as pltpu
from jax.experimental.pallas import tpu_sc as plsc

NC = 2
NS = 16
NW = NC * NS

B = 128
CH = 80
EW = B * CH
EP = NW * EW
NP = 10240

_BN_SCALE = 1.0 / math.sqrt(1.0 + 1e-5)


def _dot(a, b):
    return jnp.dot(a, b, preferred_element_type=jnp.float32)


def _node_pre_body(x_ref, w_ref, wt_ref, h_ref, a0_ref):
    h = jax.nn.relu(_dot(x_ref[...], w_ref[...]))
    h_ref[...] = h
    a0_ref[...] = _dot(h, wt_ref[...])


def _node_pre(xp, Wn0, Wfe0t):
    H2 = Wn0.shape[1]
    H = Wfe0t.shape[1]
    return pl.pallas_call(
        _node_pre_body,
        out_shape=[
            jax.ShapeDtypeStruct((NP, H2), jnp.float32),
            jax.ShapeDtypeStruct((NP, H), jnp.float32),
        ],
    )(xp, Wn0, Wfe0t)


def _make_sc_gather(H):
    ROWS = NP // NS
    mesh = plsc.VectorSubcoreMesh(core_axis_name="c", subcore_axis_name="s")

    @functools.partial(
        pl.kernel,
        mesh=mesh,
        out_type=(
            jax.ShapeDtypeStruct((NW * CH, B, H), jnp.float32),
            jax.ShapeDtypeStruct((NW * CH, B, H), jnp.float32),
        ),
        scratch_types=[
            pltpu.MemorySpace.VMEM_SHARED((NP, H), jnp.float32),
            pltpu.VMEM((CH, B), jnp.int32),
            pltpu.VMEM((2, B, H), jnp.float32),
            pltpu.SemaphoreType.DMA,
            pltpu.SemaphoreType.DMA,
        ],
    )
    def gather_k(a0_hbm, src_hbm, dst_hbm, gS_hbm, gD_hbm,
                 a0sh, idx_v, bufs, gsem, wsem):
        s = lax.axis_index("s")
        wid = lax.axis_index("c") * NS + s
        rs = pl.ds(s * ROWS, ROWS)
        pltpu.sync_copy(a0_hbm.at[rs], a0sh.at[rs])
        plsc.subcore_barrier()
        for ix_hbm, out_hbm in ((src_hbm, gS_hbm), (dst_hbm, gD_hbm)):
            pltpu.sync_copy(ix_hbm.at[wid], idx_v)
            wdesc = [None, None]
            for j in range(CH):
                b = j % 2
                if wdesc[b] is not None:
                    wdesc[b].wait()
                pltpu.async_copy(
                    a0sh.at[idx_v.at[j]], bufs.at[b], gsem).wait()
                wdesc[b] = pltpu.async_copy(
                    bufs.at[b], out_hbm.at[wid * CH + j], wsem)
            for d in wdesc:
                if d is not None:
                    d.wait()

    return gather_k


def _edge_mid_body(ea_ref, gs_ref, gd_ref, we0_ref, wb_ref, bfe_ref,
                   we1_ref, c01_ref):
    bf = jnp.bfloat16
    c0 = jax.nn.relu(_dot(ea_ref[...].astype(bf), we0_ref[...].astype(bf)))
    t = jax.nn.relu(gs_ref[...] + gd_ref[...]
                    + _dot(c0.astype(bf), wb_ref[...].astype(bf))
                    + bfe_ref[...])
    c01_ref[...] = jnp.concatenate(
        [c0, jax.nn.relu(_dot(t.astype(bf), we1_ref[...].astype(bf)))],
        axis=-1)


def _edge_mid(eap, gS, gD, We0, Wfe0, bfe0, We1):
    DE = eap.shape[1]
    H2 = We0.shape[1]
    H = Wfe0.shape[1]
    BE = 2560
    grid = (EP // BE,)
    wb = Wfe0[H2:]
    bfe = bfe0.reshape(1, H)
    full = lambda r, c: pl.BlockSpec((r, c), lambda i: (0, 0))
    return pl.pallas_call(
        _edge_mid_body,
        grid=grid,
        in_specs=[
            pl.BlockSpec((BE, DE), lambda i: (i, 0)),
            pl.BlockSpec((BE, H), lambda i: (i, 0)),
            pl.BlockSpec((BE, H), lambda i: (i, 0)),
            full(DE, H2),
            full(H2, H),
            full(1, H),
            full(H, H2),
        ],
        out_specs=pl.BlockSpec((BE, 2 * H2), lambda i: (i, 0)),
        out_shape=jax.ShapeDtypeStruct((EP, 2 * H2), jnp.float32),
    )(eap, gS, gD, We0, wb, bfe, We1)


def _make_sc_scatter(H):
    ROWS = NP // NS
    mesh = plsc.VectorSubcoreMesh(core_axis_name="c", subcore_axis_name="s")

    @functools.partial(
        pl.kernel,
        mesh=mesh,
        out_type=jax.ShapeDtypeStruct((NC, NP, H), jnp.float32),
        scratch_types=[
            pltpu.VMEM((CH, B), jnp.int32),
            pltpu.VMEM((2, B, H), jnp.float32),
            pltpu.MemorySpace.VMEM_SHARED((NP, H), jnp.float32),
            pltpu.SemaphoreType.DMA,
            pltpu.SemaphoreType.DMA,
        ],
    )
    def scatter_k(c01_hbm, dst_hbm, z_hbm, acc_hbm, idx_v, bufs, acc,
                  lsem, ssem):
        c = lax.axis_index("c")
        s = lax.axis_index("s")
        wid = c * NS + s
        rs = pl.ds(s * ROWS, ROWS)
        pltpu.sync_copy(dst_hbm.at[wid], idx_v)
        pltpu.sync_copy(z_hbm.at[rs], acc.at[rs])
        plsc.subcore_barrier()
        sdesc = [None, None]
        for j in range(CH):
            b = j % 2
            if sdesc[b] is not None:
                sdesc[b].wait()
            pltpu.async_copy(
                c01_hbm.at[wid * CH + j], bufs.at[b], lsem).wait()
            sdesc[b] = pltpu.async_copy(
                bufs.at[b], acc.at[idx_v.at[j]], ssem, add=True)
        for d in sdesc:
            if d is not None:
                d.wait()
        plsc.subcore_barrier()
        pltpu.sync_copy(acc.at[rs], acc_hbm.at[c, rs])

    return scatter_k


def _node_final_body(ap_ref, h0_ref, w0t_ref, w0b_ref, b0_ref,
                     wn1_ref, w1t_ref, w1b_ref, b1_ref, out_ref):
    H2 = h0_ref.shape[1]
    accs = ap_ref[0] + ap_ref[1]
    aggr0 = accs[:, :H2]
    aggr1 = accs[:, H2:]
    x1 = jax.nn.relu(_dot(aggr0, w0t_ref[...]) + _dot(h0_ref[...], w0b_ref[...])
                     + b0_ref[...]) * _BN_SCALE
    h1 = jax.nn.relu(_dot(x1, wn1_ref[...]))
    out_ref[...] = jax.nn.relu(
        _dot(aggr1, w1t_ref[...]) + _dot(h1, w1b_ref[...])
        + b1_ref[...]) * _BN_SCALE


def _node_final(ap, h0p, Wfn0, bfn0, Wn1, Wfn1, bfn1):
    H2 = Wn1.shape[1]
    H = Wfn0.shape[1]
    return pl.pallas_call(
        _node_final_body,
        out_shape=jax.ShapeDtypeStruct((NP, H), jnp.float32),
    )(ap, h0p, Wfn0[:H2], Wfn0[H2:], bfn0.reshape(1, H),
      Wn1, Wfn1[:H2], Wfn1[H2:], bfn1.reshape(1, H))


def kernel(x, edge_index, edge_attr, Wn0, We0, Wfn0, bfn0, Wfe0, bfe0,
           Wn1, We1, Wfn1, bfn1, Wfe1, bfe1):
    N = x.shape[0]
    E = edge_index.shape[1]
    H2 = Wn0.shape[1]
    H = Wfn0.shape[1]
    DE = edge_attr.shape[1]

    xp = jnp.pad(x, ((0, NP - N), (0, 0)))
    pad_idx = jnp.full((2, EP - E), N, dtype=edge_index.dtype)
    ei = jnp.concatenate([edge_index, pad_idx], axis=1)
    src3 = ei[0].reshape(NW, CH, B)
    dst3 = ei[1].reshape(NW, CH, B)
    eap = jnp.pad(edge_attr, ((0, EP - E), (0, 0)))

    h0p, a0 = _node_pre(xp, Wn0, Wfe0[:H2])

    gS4, gD4 = _make_sc_gather(H)(a0, src3, dst3)
    gS = gS4.reshape(EP, H)
    gD = gD4.reshape(EP, H)

    c01 = _edge_mid(eap, gS, gD, We0, Wfe0, bfe0, We1)

    zeros = jnp.zeros((NP, H), jnp.float32)
    ap = _make_sc_scatter(H)(c01.reshape(NW * CH, B, H), dst3, zeros)

    out = _node_final(ap, h0p, Wfn0, bfn0, Wn1, Wfn1, bfn1)
    return out[:N]
